# baseline (speedup 1.0000x reference)
import jax
import jax.numpy as jnp
from jax.experimental import pallas as pl
from jax.experimental.pallas import tpu as pltpu

B, N, M = 4, 4096, 4096
TN = 1024
NI = N // TN
LANES = 128


def _chamfer_kernel(x1_ref, x2t_ref, d1_ref, d2_ref, acc1_ref):
    i = pl.program_id(1)

    x1 = x1_ref[0]
    x2t = x2t_ref[0]

    inner = jax.lax.dot_general(
        x1, x2t, (((1,), (0,)), ((), ())),
        preferred_element_type=jnp.float32)
    sq1 = jnp.sum(x1 * x1, axis=1, keepdims=True)
    sq2 = 0.25 * jnp.sum(x2t * x2t, axis=0, keepdims=True)

    d = (sq1 + sq2) + inner

    fold = d[:, 0:LANES]
    for g in range(1, M // LANES):
        fold = jnp.minimum(fold, d[:, g * LANES:(g + 1) * LANES])
    acc1_ref[pl.ds(i * TN, TN), :] = fold

    part2 = jnp.min(d, axis=0)

    @pl.when(i == 0)
    def _init2():
        d2_ref[0, 0] = part2

    @pl.when(i != 0)
    def _acc2():
        d2_ref[0, 0] = jnp.minimum(d2_ref[0, 0], part2)

    @pl.when(i == NI - 1)
    def _finish():
        acc1_t = jnp.transpose(acc1_ref[:, :], (1, 0))
        d1_ref[0, 0] = jnp.maximum(jnp.min(acc1_t, axis=0), 0.0)
        d2_ref[0, 0] = jnp.maximum(d2_ref[0, 0], 0.0)


@jax.jit
def kernel(input1, input2):
    x2t = -2.0 * jnp.transpose(input2, (0, 2, 1))

    grid = (B, NI)
    dist1, dist2 = pl.pallas_call(
        _chamfer_kernel,
        grid=grid,
        in_specs=[
            pl.BlockSpec((1, TN, 3), lambda b, i: (b, i, 0)),
            pl.BlockSpec((1, 3, M), lambda b, i: (b, 0, 0)),
        ],
        out_specs=[
            pl.BlockSpec((1, 1, N), lambda b, i: (b, 0, 0)),
            pl.BlockSpec((1, 1, M), lambda b, i: (b, 0, 0)),
        ],
        out_shape=[
            jax.ShapeDtypeStruct((B, 1, N), jnp.float32),
            jax.ShapeDtypeStruct((B, 1, M), jnp.float32),
        ],
        scratch_shapes=[pltpu.VMEM((N, LANES), jnp.float32)],
        compiler_params=pltpu.CompilerParams(
            dimension_semantics=("parallel", "arbitrary")),
    )(input1, x2t)
    dist1 = dist1.reshape(B, N)
    dist2 = dist2.reshape(B, M)
    loss = jnp.mean(dist1) + jnp.mean(dist2)
    return (loss, dist1, dist2)
